# Initial kernel scaffold; baseline (speedup 1.0000x reference)
import functools

import jax
import jax.numpy as jnp
from jax import lax
from jax.experimental import pallas as pl
from jax.experimental.pallas import tpu as pltpu
from jax.experimental.pallas import tpu_sc as plsc

N = 10000
E = 320000
D = 128
H = 128

NC = 2
NS = 16
NW = NC * NS

NP = 10240
RB = 320
NB = NP // RB
CH = 128
EPW = E // NW
EPWP = 10240
NCHUNK = EPWP // CH
ROWS_PER_TILE = NP // NS

_mesh = plsc.VectorSubcoreMesh(
    core_axis_name="c", subcore_axis_name="s", num_cores=NC, num_subcores=NS
)


def _hist_body(dst_hbm, hist_hbm, idx_v, hist_v):
    cid = lax.axis_index("c")
    sid = lax.axis_index("s")
    wid = sid * NC + cid
    pltpu.sync_copy(dst_hbm.at[pl.ds(wid * EPW, EPW)], idx_v)

    @pl.loop(0, NP // 16)
    def _zero(i):
        hist_v[pl.ds(i * 16, 16)] = jnp.zeros((16,), jnp.float32)

    ones = jnp.ones((16,), jnp.float32)

    @pl.loop(0, EPW // 16)
    def _count(i):
        ids = idx_v[pl.ds(i * 16, 16)]
        plsc.addupdate_scatter(hist_v, [ids], ones)

    pltpu.sync_copy(hist_v, hist_hbm.at[wid])


_hist_call = functools.partial(
    pl.kernel,
    out_type=jax.ShapeDtypeStruct((NW, NP), jnp.float32),
    mesh=_mesh,
    scratch_types=[
        pltpu.VMEM((EPW,), jnp.int32),
        pltpu.VMEM((NP,), jnp.float32),
    ],
)(_hist_body)


def _disg_kernel(hist_ref, x_ref, w1_ref, b1_ref, g_ref):
    bi = pl.program_id(0)
    deg = jnp.sum(hist_ref[...], axis=0) + 1.0
    dis = lax.rsqrt(deg)
    h = jnp.dot(x_ref[...], w1_ref[...], preferred_element_type=jnp.float32)
    h = h + b1_ref[...]
    rows = lax.broadcasted_iota(jnp.int32, (RB, 1), 0) + bi * RB
    g_ref[...] = jnp.where(rows < N, h * dis[:, None], 0.0)


_disg_call = pl.pallas_call(
    _disg_kernel,
    grid=(NB,),
    in_specs=[
        pl.BlockSpec((NW, RB), lambda i: (0, i)),
        pl.BlockSpec((RB, D), lambda i: (i, 0)),
        pl.BlockSpec((D, H), lambda i: (0, 0)),
        pl.BlockSpec((1, H), lambda i: (0, 0)),
    ],
    out_specs=pl.BlockSpec((RB, H), lambda i: (i, 0)),
    out_shape=jax.ShapeDtypeStruct((NP, H), jnp.float32),
)


def _agg_body(g_hbm, src_hbm, dst_hbm, out_hbm,
              idxs, idxd, buf0, buf1, agg, sg0, sg1):
    cid = lax.axis_index("c")
    sid = lax.axis_index("s")
    wid = sid * NC + cid

    pltpu.sync_copy(src_hbm.at[wid], idxs)
    pltpu.sync_copy(dst_hbm.at[wid], idxd)

    @pl.loop(0, CH)
    def _zrow(i):
        for v in range(H // 16):
            buf0[i, pl.ds(v * 16, 16)] = jnp.zeros((16,), jnp.float32)

    for z in range(ROWS_PER_TILE // CH):
        pltpu.sync_copy(buf0, agg.at[pl.ds(sid * ROWS_PER_TILE + z * CH, CH)])
    plsc.subcore_barrier()

    bufs = (buf0, buf1)
    sems = (sg0, sg1)
    pltpu.async_copy(g_hbm.at[idxs.at[0]], buf0, sg0)
    pltpu.async_copy(g_hbm.at[idxs.at[1]], buf1, sg1)

    @pl.loop(0, NCHUNK - 2, step=2)
    def _main(j0):
        for b in range(2):
            j = j0 + b
            pltpu.make_async_copy(g_hbm.at[idxs.at[j]], bufs[b], sems[b]).wait()
            pltpu.sync_copy(bufs[b], agg.at[idxd.at[j]], add=True)
            pltpu.async_copy(g_hbm.at[idxs.at[j + 2]], bufs[b], sems[b])

    for b in range(2):
        j = NCHUNK - 2 + b
        pltpu.make_async_copy(g_hbm.at[idxs.at[j]], bufs[b], sems[b]).wait()
        pltpu.sync_copy(bufs[b], agg.at[idxd.at[j]], add=True)

    plsc.subcore_barrier()
    pltpu.sync_copy(
        agg.at[pl.ds(sid * ROWS_PER_TILE, ROWS_PER_TILE)],
        out_hbm.at[cid, pl.ds(sid * ROWS_PER_TILE, ROWS_PER_TILE)],
    )


_agg_call = functools.partial(
    pl.kernel,
    out_type=jax.ShapeDtypeStruct((NC, NP, H), jnp.float32),
    mesh=_mesh,
    scratch_types=[
        pltpu.VMEM((NCHUNK, CH), jnp.int32),
        pltpu.VMEM((NCHUNK, CH), jnp.int32),
        pltpu.VMEM((CH, H), jnp.float32),
        pltpu.VMEM((CH, H), jnp.float32),
        pltpu.VMEM_SHARED((NP, H), jnp.float32),
        pltpu.SemaphoreType.DMA,
        pltpu.SemaphoreType.DMA,
    ],
)(_agg_body)


def _out_kernel(agg_ref, g_ref, hist_ref, w2t_ref, b2_ref, out_ref):
    deg = jnp.sum(hist_ref[...], axis=0) + 1.0
    dis = lax.rsqrt(deg)
    t = (agg_ref[0] + agg_ref[1] + g_ref[...]) * dis[:, None]
    t = jnp.maximum(t, 0.0)
    out_ref[...] = jnp.sum(t * w2t_ref[...], axis=1, keepdims=True) + b2_ref[0, 0]


_out_call = pl.pallas_call(
    _out_kernel,
    grid=(NB,),
    in_specs=[
        pl.BlockSpec((NC, RB, H), lambda i: (0, i, 0)),
        pl.BlockSpec((RB, H), lambda i: (i, 0)),
        pl.BlockSpec((NW, RB), lambda i: (0, i)),
        pl.BlockSpec((1, H), lambda i: (0, 0)),
        pl.BlockSpec((1, 1), lambda i: (0, 0)),
    ],
    out_specs=pl.BlockSpec((RB, 1), lambda i: (i, 0)),
    out_shape=jax.ShapeDtypeStruct((NP, 1), jnp.float32),
)


def kernel(x, edge_index, W1, b1, W2, b2):
    src = edge_index[0].astype(jnp.int32)
    dst = edge_index[1].astype(jnp.int32)
    srcp = jnp.pad(
        src.reshape(NW, EPW), ((0, 0), (0, EPWP - EPW)), constant_values=N
    ).reshape(NW, NCHUNK, CH)
    dstp = jnp.pad(
        dst.reshape(NW, EPW), ((0, 0), (0, EPWP - EPW)), constant_values=N
    ).reshape(NW, NCHUNK, CH)
    xp = jnp.pad(x, ((0, NP - N), (0, 0)))

    hist = _hist_call(dst)
    g = _disg_call(hist, xp, W1, b1.reshape(1, H))
    agg = _agg_call(g, srcp, dstp)
    outp = _out_call(agg, g, hist, W2.reshape(1, H), b2.reshape(1, 1))
    return outp[:N]


# speedup vs baseline: 14.6754x; 14.6754x over previous
import functools

import jax
import jax.numpy as jnp
from jax import lax
from jax.experimental import pallas as pl
from jax.experimental.pallas import tpu as pltpu
from jax.experimental.pallas import tpu_sc as plsc

N = 10000
E = 320000
D = 128
H = 128

NC = 2
NS = 16
NW = NC * NS

NP = 10240
RB = 128
NB = NP // RB
CH = 128
EPW = E // NW
EPWP = 10240
NCHUNK = EPWP // CH
HALF = NCHUNK // 2
ROWS_PER_TILE = NP // NS

_mesh = plsc.VectorSubcoreMesh(
    core_axis_name="c", subcore_axis_name="s", num_cores=NC, num_subcores=NS
)


def _hist_body(dst_hbm, hist_hbm, idx_v, hist_v):
    cid = lax.axis_index("c")
    sid = lax.axis_index("s")
    wid = sid * NC + cid
    pltpu.sync_copy(dst_hbm.at[pl.ds(wid * EPW, EPW)], idx_v)

    @pl.loop(0, NP // 16)
    def _zero(i):
        hist_v[pl.ds(i * 16, 16)] = jnp.zeros((16,), jnp.float32)

    ones = jnp.ones((16,), jnp.float32)

    @pl.loop(0, EPW // 16)
    def _count(i):
        ids = idx_v[pl.ds(i * 16, 16)]
        plsc.addupdate_scatter(hist_v, [ids], ones)

    pltpu.sync_copy(hist_v, hist_hbm.at[wid])


_hist_call = functools.partial(
    pl.kernel,
    out_type=jax.ShapeDtypeStruct((NW, NP), jnp.float32),
    mesh=_mesh,
    compiler_params=pltpu.CompilerParams(needs_layout_passes=False),
    scratch_types=[
        pltpu.VMEM((EPW,), jnp.int32),
        pltpu.VMEM((NP,), jnp.float32),
    ],
)(_hist_body)


def _disg_kernel(hist_ref, x_ref, w1_ref, b1_ref, g_ref):
    bi = pl.program_id(0)
    deg = jnp.sum(hist_ref[...], axis=0) + 1.0
    dis = lax.rsqrt(deg)
    h = jnp.dot(x_ref[...], w1_ref[...], preferred_element_type=jnp.float32)
    h = h + b1_ref[...]
    rows = lax.broadcasted_iota(jnp.int32, (RB, 1), 0) + bi * RB
    g_ref[...] = jnp.where(rows < N, h * dis[:, None], 0.0)


_disg_call = pl.pallas_call(
    _disg_kernel,
    grid=(NB,),
    in_specs=[
        pl.BlockSpec((NW, RB), lambda i: (0, i)),
        pl.BlockSpec((RB, D), lambda i: (i, 0)),
        pl.BlockSpec((D, H), lambda i: (0, 0)),
        pl.BlockSpec((1, H), lambda i: (0, 0)),
    ],
    out_specs=pl.BlockSpec((RB, H), lambda i: (i, 0)),
    out_shape=jax.ShapeDtypeStruct((NP, H), jnp.float32),
)


def _agg_body(g_hbm, src_hbm, dst_hbm, out_hbm,
              idxs, idxd, buf0, buf1, agg, sg0, sg1):
    cid = lax.axis_index("c")
    sid = lax.axis_index("s")
    wid = sid * NC + cid

    @pl.loop(0, CH)
    def _zrow(i):
        for v in range(H // 16):
            buf0[i, pl.ds(v * 16, 16)] = jnp.zeros((16,), jnp.float32)

    for z in range(ROWS_PER_TILE // CH):
        pltpu.sync_copy(buf0, agg.at[pl.ds(sid * ROWS_PER_TILE + z * CH, CH)])
    plsc.subcore_barrier()

    bufs = (buf0, buf1)
    sems = (sg0, sg1)
    for h in range(2):
        pltpu.sync_copy(src_hbm.at[wid, pl.ds(h * HALF, HALF)], idxs)
        pltpu.sync_copy(dst_hbm.at[wid, pl.ds(h * HALF, HALF)], idxd)
        pltpu.async_copy(g_hbm.at[idxs.at[0]], buf0, sg0)
        pltpu.async_copy(g_hbm.at[idxs.at[1]], buf1, sg1)

        @pl.loop(0, HALF - 2, step=2)
        def _main(j0):
            for b in range(2):
                j = j0 + b
                pltpu.make_async_copy(g_hbm.at[idxs.at[j]], bufs[b], sems[b]).wait()
                pltpu.sync_copy(bufs[b], agg.at[idxd.at[j]], add=True)
                pltpu.async_copy(g_hbm.at[idxs.at[j + 2]], bufs[b], sems[b])

        for b in range(2):
            j = HALF - 2 + b
            pltpu.make_async_copy(g_hbm.at[idxs.at[j]], bufs[b], sems[b]).wait()
            pltpu.sync_copy(bufs[b], agg.at[idxd.at[j]], add=True)

    plsc.subcore_barrier()
    pltpu.sync_copy(
        agg.at[pl.ds(sid * ROWS_PER_TILE, ROWS_PER_TILE)],
        out_hbm.at[cid, pl.ds(sid * ROWS_PER_TILE, ROWS_PER_TILE)],
    )


_agg_call = functools.partial(
    pl.kernel,
    out_type=jax.ShapeDtypeStruct((NC, NP, H), jnp.float32),
    mesh=_mesh,
    compiler_params=pltpu.CompilerParams(needs_layout_passes=False),
    scratch_types=[
        pltpu.VMEM((HALF, CH), jnp.int32),
        pltpu.VMEM((HALF, CH), jnp.int32),
        pltpu.VMEM((CH, H), jnp.float32),
        pltpu.VMEM((CH, H), jnp.float32),
        pltpu.VMEM_SHARED((NP, H), jnp.float32),
        pltpu.SemaphoreType.DMA,
        pltpu.SemaphoreType.DMA,
    ],
)(_agg_body)


def _out_kernel(agg_ref, g_ref, hist_ref, w2t_ref, b2_ref, out_ref):
    deg = jnp.sum(hist_ref[...], axis=0) + 1.0
    dis = lax.rsqrt(deg)
    t = (agg_ref[0] + agg_ref[1] + g_ref[...]) * dis[:, None]
    t = jnp.maximum(t, 0.0)
    out_ref[...] = jnp.sum(t * w2t_ref[...], axis=1, keepdims=True) + b2_ref[0, 0]


_out_call = pl.pallas_call(
    _out_kernel,
    grid=(NB,),
    in_specs=[
        pl.BlockSpec((NC, RB, H), lambda i: (0, i, 0)),
        pl.BlockSpec((RB, H), lambda i: (i, 0)),
        pl.BlockSpec((NW, RB), lambda i: (0, i)),
        pl.BlockSpec((1, H), lambda i: (0, 0)),
        pl.BlockSpec((1, 1), lambda i: (0, 0)),
    ],
    out_specs=pl.BlockSpec((RB, 1), lambda i: (i, 0)),
    out_shape=jax.ShapeDtypeStruct((NP, 1), jnp.float32),
)


def kernel(x, edge_index, W1, b1, W2, b2):
    src = edge_index[0].astype(jnp.int32)
    dst = edge_index[1].astype(jnp.int32)
    srcp = jnp.pad(
        src.reshape(NW, EPW), ((0, 0), (0, EPWP - EPW)), constant_values=N
    ).reshape(NW, NCHUNK, CH)
    dstp = jnp.pad(
        dst.reshape(NW, EPW), ((0, 0), (0, EPWP - EPW)), constant_values=N
    ).reshape(NW, NCHUNK, CH)
    xp = jnp.pad(x, ((0, NP - N), (0, 0)))

    hist = _hist_call(dst)
    g = _disg_call(hist, xp, W1, b1.reshape(1, H))
    agg = _agg_call(g, srcp, dstp)
    outp = _out_call(agg, g, hist, W2.reshape(1, H), b2.reshape(1, 1))
    return outp[:N]
